# Initial kernel scaffold; baseline (speedup 1.0000x reference)
import jax
import jax.numpy as jnp
from jax.experimental import pallas as pl


def kernel(x, score, emb_table, W, b):
    raise NotImplementedError("write your pallas kernel here")


# speedup vs baseline: 9.1320x; 9.1320x over previous
import functools

import jax
import jax.numpy as jnp
from jax import lax
from jax.experimental import pallas as pl
from jax.experimental.pallas import tpu as pltpu
from jax.experimental.pallas import tpu_sc as plsc

N_TOK = 100000
D = 128
B = 4096
H = 50

MV_BLK = 4096
MV_GRID = 25
N_PAD = MV_BLK * MV_GRID

NC = 2
NS = 16
NW = NC * NS
CHUNK = (B * H) // NW
STEPS = CHUNK // 16
ACC = 64


def _matvec_body(w_ref, e_ref, o_ref):
    o_ref[...] = lax.dot_general(
        w_ref[...], e_ref[...], (((1,), (1,)), ((), ())),
        precision=lax.Precision.HIGHEST, preferred_element_type=jnp.float32)


def _matvec(emb_table, W):
    return pl.pallas_call(
        _matvec_body,
        grid=(MV_GRID,),
        in_specs=[
            pl.BlockSpec((1, D), lambda i: (0, 0)),
            pl.BlockSpec((MV_BLK, D), lambda i: (i, 0)),
        ],
        out_specs=pl.BlockSpec((1, MV_BLK), lambda i: (0, i)),
        out_shape=jax.ShapeDtypeStruct((1, N_PAD), jnp.float32),
    )(W, emb_table)


def _pool_body(t_hbm, x_hbm, s_hbm, out_hbm, t_v, x_v, s_v, acc_v):
    wid = lax.axis_index("s") * NC + lax.axis_index("c")
    base = wid * CHUNK
    pltpu.sync_copy(t_hbm, t_v)
    pltpu.sync_copy(x_hbm.at[pl.ds(base, CHUNK)], x_v)
    pltpu.sync_copy(s_hbm.at[pl.ds(base, CHUNK)], s_v)
    for j in range(ACC // 16):
        acc_v[pl.ds(j * 16, 16)] = jnp.zeros((16,), jnp.float32)

    def body(i, carry):
        off = i * 16
        idx = x_v[pl.ds(off, 16)]
        sv = s_v[pl.ds(off, 16)]
        vals = plsc.load_gather(t_v, [idx])
        lidx = lax.rem(off + lax.iota(jnp.int32, 16), jnp.int32(H))
        plsc.addupdate_scatter(acc_v, [lidx], vals * sv)
        return carry

    lax.fori_loop(0, STEPS, body, 0)
    pltpu.sync_copy(acc_v, out_hbm.at[wid])


def _pool(t, x_flat, s_flat):
    mesh = plsc.VectorSubcoreMesh(
        core_axis_name="c", subcore_axis_name="s",
        num_cores=NC, num_subcores=NS)
    run = pl.kernel(
        _pool_body,
        out_type=jax.ShapeDtypeStruct((NW, ACC), jnp.float32),
        mesh=mesh,
        compiler_params=pltpu.CompilerParams(needs_layout_passes=False),
        scratch_types=[
            pltpu.VMEM((N_PAD,), jnp.float32),
            pltpu.VMEM((CHUNK,), jnp.int32),
            pltpu.VMEM((CHUNK,), jnp.float32),
            pltpu.VMEM((ACC,), jnp.float32),
        ],
    )
    return run(t, x_flat, s_flat)


def _combine_body(p_ref, b_ref, o_ref):
    pooled = jnp.sum(p_ref[...], axis=0, keepdims=True) * (1.0 / B)
    o_ref[...] = jax.nn.sigmoid(pooled + b_ref[0, 0])


def _combine(partials, b):
    return pl.pallas_call(
        _combine_body,
        in_specs=[
            pl.BlockSpec((NW, ACC), lambda: (0, 0)),
            pl.BlockSpec((1, 1), lambda: (0, 0)),
        ],
        out_specs=pl.BlockSpec((1, ACC), lambda: (0, 0)),
        out_shape=jax.ShapeDtypeStruct((1, ACC), jnp.float32),
    )(partials, b.reshape(1, 1))


def kernel(x, score, emb_table, W, b):
    t = _matvec(emb_table, W).reshape(N_PAD)
    x_flat = x.reshape(-1).astype(jnp.int32)
    s_flat = score.reshape(-1)
    partials = _pool(t, x_flat, s_flat)
    out = _combine(partials, b)
    return out[0, :H].reshape(H, 1)


# speedup vs baseline: 11.5500x; 1.2648x over previous
import functools

import jax
import jax.numpy as jnp
from jax import lax
from jax.experimental import pallas as pl
from jax.experimental.pallas import tpu as pltpu
from jax.experimental.pallas import tpu_sc as plsc

N_TOK = 100000
D = 128
B = 4096
H = 50

MV_BLK = 4096
MV_GRID = 25
N_PAD = MV_BLK * MV_GRID

NC = 2
NS = 16
NW = NC * NS
CHUNK = (B * H) // NW
STEPS = CHUNK // 16
ACC = 64


def _matvec_body(w_ref, e_ref, o_ref):
    o_ref[...] = lax.dot_general(
        w_ref[...].astype(jnp.bfloat16), e_ref[...].astype(jnp.bfloat16),
        (((1,), (1,)), ((), ())), preferred_element_type=jnp.float32)


def _matvec(emb_table, W):
    return pl.pallas_call(
        _matvec_body,
        grid=(MV_GRID,),
        in_specs=[
            pl.BlockSpec((1, D), lambda i: (0, 0)),
            pl.BlockSpec((MV_BLK, D), lambda i: (i, 0)),
        ],
        out_specs=pl.BlockSpec((1, MV_BLK), lambda i: (0, i)),
        out_shape=jax.ShapeDtypeStruct((1, N_PAD), jnp.float32),
    )(W, emb_table)


def _pool_body(t_hbm, x_hbm, s_hbm, out_hbm, t_v, x_v, s_v, acc_v):
    wid = lax.axis_index("s") * NC + lax.axis_index("c")
    base = wid * CHUNK
    pltpu.sync_copy(t_hbm, t_v)
    pltpu.sync_copy(x_hbm.at[pl.ds(base, CHUNK)], x_v)
    pltpu.sync_copy(s_hbm.at[pl.ds(base, CHUNK)], s_v)
    for j in range(ACC // 16):
        acc_v[pl.ds(j * 16, 16)] = jnp.zeros((16,), jnp.float32)

    def body(i, carry):
        off = i * 16
        idx = x_v[pl.ds(off, 16)]
        sv = s_v[pl.ds(off, 16)]
        vals = plsc.load_gather(t_v, [idx])
        lidx = lax.rem(off + lax.iota(jnp.int32, 16), jnp.int32(H))
        plsc.addupdate_scatter(acc_v, [lidx], vals * sv)
        return carry

    lax.fori_loop(0, STEPS, body, 0)
    pltpu.sync_copy(acc_v, out_hbm.at[wid])


def _pool(t, x_flat, s_flat):
    mesh = plsc.VectorSubcoreMesh(
        core_axis_name="c", subcore_axis_name="s",
        num_cores=NC, num_subcores=NS)
    run = pl.kernel(
        _pool_body,
        out_type=jax.ShapeDtypeStruct((NW, ACC), jnp.float32),
        mesh=mesh,
        compiler_params=pltpu.CompilerParams(needs_layout_passes=False),
        scratch_types=[
            pltpu.VMEM((N_PAD,), jnp.float32),
            pltpu.VMEM((CHUNK,), jnp.int32),
            pltpu.VMEM((CHUNK,), jnp.float32),
            pltpu.VMEM((ACC,), jnp.float32),
        ],
    )
    return run(t, x_flat, s_flat)


def _combine_body(p_ref, b_ref, o_ref):
    pooled = jnp.sum(p_ref[...], axis=0, keepdims=True) * (1.0 / B)
    o_ref[...] = jax.nn.sigmoid(pooled + b_ref[0, 0])


def _combine(partials, b):
    return pl.pallas_call(
        _combine_body,
        in_specs=[
            pl.BlockSpec((NW, ACC), lambda: (0, 0)),
            pl.BlockSpec((1, 1), lambda: (0, 0)),
        ],
        out_specs=pl.BlockSpec((1, ACC), lambda: (0, 0)),
        out_shape=jax.ShapeDtypeStruct((1, ACC), jnp.float32),
    )(partials, b.reshape(1, 1))


def kernel(x, score, emb_table, W, b):
    t = _matvec(emb_table, W).reshape(N_PAD)
    x_flat = x.reshape(-1).astype(jnp.int32)
    s_flat = score.reshape(-1)
    partials = _pool(t, x_flat, s_flat)
    out = _combine(partials, b)
    return out[0, :H].reshape(H, 1)


# speedup vs baseline: 13.1522x; 1.1387x over previous
import functools

import jax
import jax.numpy as jnp
from jax import lax
from jax.experimental import pallas as pl
from jax.experimental.pallas import tpu as pltpu
from jax.experimental.pallas import tpu_sc as plsc

N_TOK = 100000
D = 128
B = 4096
H = 50

MV_BLK = 8192
MV_GRID = 13
N_PAD = MV_BLK * MV_GRID

NC = 2
NS = 16
NW = NC * NS
CHUNK = (B * H) // NW
RPW = B // NW
STEPS = CHUNK // 16
ACC = 64


def _matvec_body(w_ref, e_ref, o_ref):
    r = lax.dot_general(
        w_ref[...].astype(jnp.bfloat16), e_ref[...].astype(jnp.bfloat16),
        (((1,), (1,)), ((), ())), preferred_element_type=jnp.float32)
    o_ref[...] = r.reshape(MV_BLK)


def _matvec(emb_table, W):
    return pl.pallas_call(
        _matvec_body,
        grid=(MV_GRID,),
        in_specs=[
            pl.BlockSpec((1, D), lambda i: (0, 0)),
            pl.BlockSpec((MV_BLK, D), lambda i: (i, 0)),
        ],
        out_specs=pl.BlockSpec((MV_BLK,), lambda i: (i,)),
        out_shape=jax.ShapeDtypeStruct((N_PAD,), jnp.float32),
    )(W, emb_table)


def _pool_body(t_hbm, xs_hbm, out_hbm, t_v, x_v, s_v, acc_v,
               sem_t, sem_x, sem_s):
    wid = lax.axis_index("s") * NC + lax.axis_index("c")
    base = wid * CHUNK
    ct = pltpu.async_copy(t_hbm, t_v, sem_t)
    cx = pltpu.async_copy(xs_hbm.at[pl.ds(base, CHUNK)], x_v, sem_x)
    cs = pltpu.async_copy(xs_hbm.at[pl.ds(B * H + base, CHUNK)], s_v, sem_s)
    for j in range(ACC // 16):
        acc_v[pl.ds(j * 16, 16)] = jnp.zeros((16,), jnp.float32)
    cx.wait()
    cs.wait()
    ct.wait()

    def body(i, carry):
        off = i * 16
        idx = x_v[pl.ds(off, 16)]
        sv = plsc.bitcast(s_v[pl.ds(off, 16)], jnp.float32)
        vals = plsc.load_gather(t_v, [idx])
        lidx = lax.rem(off + lax.iota(jnp.int32, 16), jnp.int32(H))
        plsc.addupdate_scatter(acc_v, [lidx], vals * sv)
        return carry

    lax.fori_loop(0, STEPS, body, 0, unroll=4)
    pltpu.sync_copy(acc_v, out_hbm.at[wid])


def _pool(t, xs_packed):
    mesh = plsc.VectorSubcoreMesh(
        core_axis_name="c", subcore_axis_name="s",
        num_cores=NC, num_subcores=NS)
    run = pl.kernel(
        _pool_body,
        out_type=jax.ShapeDtypeStruct((NW, ACC), jnp.float32),
        mesh=mesh,
        compiler_params=pltpu.CompilerParams(needs_layout_passes=False),
        scratch_types=[
            pltpu.VMEM((N_PAD,), jnp.float32),
            pltpu.VMEM((CHUNK,), jnp.int32),
            pltpu.VMEM((CHUNK,), jnp.int32),
            pltpu.VMEM((ACC,), jnp.float32),
            pltpu.SemaphoreType.DMA,
            pltpu.SemaphoreType.DMA,
            pltpu.SemaphoreType.DMA,
        ],
    )
    return run(t, xs_packed)


def _combine_body(p_ref, b_ref, o_ref):
    pooled = jnp.sum(p_ref[...], axis=0, keepdims=True) * (1.0 / B)
    o_ref[...] = jax.nn.sigmoid(pooled + b_ref[0, 0])


def _combine(partials, b):
    return pl.pallas_call(
        _combine_body,
        in_specs=[
            pl.BlockSpec((NW, ACC), lambda: (0, 0)),
            pl.BlockSpec((1, 1), lambda: (0, 0)),
        ],
        out_specs=pl.BlockSpec((1, ACC), lambda: (0, 0)),
        out_shape=jax.ShapeDtypeStruct((1, ACC), jnp.float32),
    )(partials, b.reshape(1, 1))


def kernel(x, score, emb_table, W, b):
    t = _matvec(emb_table, W)
    xs_packed = jnp.concatenate([
        x.reshape(-1).astype(jnp.int32),
        lax.bitcast_convert_type(score.reshape(-1), jnp.int32),
    ])
    partials = _pool(t, xs_packed)
    out = _combine(partials, b)
    return out[0, :H].reshape(H, 1)
